# baseline (speedup 1.0000x reference)
import jax
import jax.numpy as jnp
from jax.experimental import pallas as pl
from jax.experimental.pallas import tpu as pltpu

_CHUNKS = 16


def _copy_body(k_ref, v_ref, ok_ref, ov_ref, sem):
    rows = k_ref.shape[0]
    cr = rows // _CHUNKS
    copies = []
    for i in range(_CHUNKS):
        sl = pl.ds(i * cr, cr)
        copies.append(pltpu.make_async_copy(k_ref.at[sl], ok_ref.at[sl], sem.at[2 * i]))
        copies.append(pltpu.make_async_copy(v_ref.at[sl], ov_ref.at[sl], sem.at[2 * i + 1]))
    for c in copies:
        c.start()
    for c in copies:
        c.wait()


def kernel(new_k, new_v):
    shape = new_k.shape
    cols = 2048
    rows = new_k.size // cols
    k2 = new_k.reshape(rows, cols)
    v2 = new_v.reshape(rows, cols)
    out2 = pl.pallas_call(
        _copy_body,
        in_specs=[
            pl.BlockSpec(memory_space=pl.ANY),
            pl.BlockSpec(memory_space=pl.ANY),
        ],
        out_specs=[
            pl.BlockSpec(memory_space=pl.ANY),
            pl.BlockSpec(memory_space=pl.ANY),
        ],
        out_shape=(
            jax.ShapeDtypeStruct((rows, cols), new_k.dtype),
            jax.ShapeDtypeStruct((rows, cols), new_v.dtype),
        ),
        scratch_shapes=[pltpu.SemaphoreType.DMA((2 * _CHUNKS,))],
    )(k2, v2)
    return (out2[0].reshape(shape), out2[1].reshape(shape))


# speedup vs baseline: 12.1284x; 12.1284x over previous
import jax
import jax.numpy as jnp
from jax.experimental import pallas as pl
from jax.experimental.pallas import tpu as pltpu

_COLS = 2048
_BLOCK_ROWS = 512


def _copy_body(k_ref, v_ref, ok_ref, ov_ref):
    ok_ref[...] = k_ref[...]
    ov_ref[...] = v_ref[...]


def kernel(new_k, new_v):
    shape = new_k.shape
    total = new_k.size
    rows = total // _COLS
    k2 = new_k.reshape(rows, _COLS)
    v2 = new_v.reshape(rows, _COLS)
    n_blocks = rows // _BLOCK_ROWS

    spec = pl.BlockSpec((_BLOCK_ROWS, _COLS), lambda i: (i, 0))
    out2 = pl.pallas_call(
        _copy_body,
        grid=(n_blocks,),
        in_specs=[spec, spec],
        out_specs=[spec, spec],
        out_shape=(
            jax.ShapeDtypeStruct((rows, _COLS), new_k.dtype),
            jax.ShapeDtypeStruct((rows, _COLS), new_v.dtype),
        ),
        compiler_params=pltpu.CompilerParams(
            dimension_semantics=("arbitrary",),
        ),
    )(k2, v2)
    return (out2[0].reshape(shape), out2[1].reshape(shape))


# speedup vs baseline: 52.0001x; 4.2875x over previous
import jax
import jax.numpy as jnp
from jax.experimental import pallas as pl
from jax.experimental.pallas import tpu as pltpu

_HBLK = 4


def _copy_body(k_ref, v_ref, ok_ref, ov_ref):
    ok_ref[...] = k_ref[...]
    ov_ref[...] = v_ref[...]


def kernel(new_k, new_v):
    B, H, L, D = new_k.shape
    nh = H // _HBLK
    spec = pl.BlockSpec((1, _HBLK, L, D), lambda b, h: (b, h, 0, 0))
    return pl.pallas_call(
        _copy_body,
        grid=(B, nh),
        in_specs=[spec, spec],
        out_specs=[spec, spec],
        out_shape=(
            jax.ShapeDtypeStruct(new_k.shape, new_k.dtype),
            jax.ShapeDtypeStruct(new_v.shape, new_v.dtype),
        ),
        compiler_params=pltpu.CompilerParams(
            dimension_semantics=("arbitrary", "arbitrary"),
        ),
    )(new_k, new_v)
